# Initial kernel scaffold; baseline (speedup 1.0000x reference)
import jax
import jax.numpy as jnp
from jax.experimental import pallas as pl


def kernel(x, edge_index, edge_attr, batch, params):
    raise NotImplementedError("write your pallas kernel here")


# speedup vs baseline: 1.5698x; 1.5698x over previous
import functools

import jax
import jax.numpy as jnp
from jax.experimental import pallas as pl

D = 128
LEAKY = 0.2


def _leaky(v):
    return jnp.where(v >= 0, v, LEAKY * v)


def _node_pre_body(x_ref, w1s_ref, w1d_ref, b1_ref, gatw_ref, asrc_ref,
                   adst_ref, xs1_ref, xd1_ref, xl_ref, av_ref):
    x = x_ref[...]
    xs1_ref[...] = x @ w1s_ref[...] + b1_ref[...]
    xd1_ref[...] = x @ w1d_ref[...]
    xl = x @ gatw_ref[...]
    xl_ref[...] = xl
    a_src = xl @ asrc_ref[...]
    a_dst = xl @ adst_ref[...]
    exloop = jnp.exp(_leaky(a_src + a_dst))
    av_ref[...] = jnp.concatenate([a_src, a_dst, exloop], axis=1)


def _node_pre(x, w1s, w1d, b1, gatw, att_src, att_dst):
    n = x.shape[0]
    out = pl.pallas_call(
        _node_pre_body,
        out_shape=[
            jax.ShapeDtypeStruct((n, D), jnp.float32),
            jax.ShapeDtypeStruct((n, D), jnp.float32),
            jax.ShapeDtypeStruct((n, D), jnp.float32),
            jax.ShapeDtypeStruct((n, 3), jnp.float32),
        ],
    )(x, w1s, w1d, b1[None, :], gatw, att_src[:, None], att_dst[:, None])
    xs1, xd1, xl, av = out
    return xs1, xd1, xl, av[:, 0], av[:, 1], av[:, 2]


def _edge_dense_body(gs_ref, ea_ref, sa_ref, w1e_ref, w2_ref, b2_ref,
                     ve_ref, lew_ref, leb_ref, eaout_ref, ex_ref):
    ea = ea_ref[...]
    t = jnp.maximum(gs_ref[...] + ea @ w1e_ref[...], 0.0)
    h = t @ w2_ref[...] + b2_ref[...]
    if ea.shape[1] == D:
        ea_new = h + ea
    else:
        ea_new = h + ea
    a_edge = ea_new @ ve_ref[...]
    alpha = _leaky(sa_ref[...] + a_edge)
    ex_ref[...] = jnp.exp(alpha)
    eaout_ref[...] = jnp.maximum(ea_new @ lew_ref[...] + leb_ref[...], 0.0)


def _edge_dense(gs, ea, sa, w1e, w2, b2, ve, lew, leb, blk=2000):
    e = gs.shape[0]
    ein = ea.shape[1]
    grid = (e // blk,)
    eaout, ex = pl.pallas_call(
        _edge_dense_body,
        grid=grid,
        in_specs=[
            pl.BlockSpec((blk, D), lambda i: (i, 0)),
            pl.BlockSpec((blk, ein), lambda i: (i, 0)),
            pl.BlockSpec((blk, 1), lambda i: (i, 0)),
            pl.BlockSpec((ein, D), lambda i: (0, 0)),
            pl.BlockSpec((D, D), lambda i: (0, 0)),
            pl.BlockSpec((1, D), lambda i: (0, 0)),
            pl.BlockSpec((D, 1), lambda i: (0, 0)),
            pl.BlockSpec((D, D), lambda i: (0, 0)),
            pl.BlockSpec((1, D), lambda i: (0, 0)),
        ],
        out_specs=[
            pl.BlockSpec((blk, D), lambda i: (i, 0)),
            pl.BlockSpec((blk, 1), lambda i: (i, 0)),
        ],
        out_shape=[
            jax.ShapeDtypeStruct((e, D), jnp.float32),
            jax.ShapeDtypeStruct((e, 1), jnp.float32),
        ],
    )(gs, ea, sa[:, None], w1e, w2, b2[None, :], ve[:, None], lew,
      leb[None, :])
    return eaout, ex[:, 0]


def _node_fin_body(num_ref, den_ref, exloop_ref, xl_ref, gatb_ref, bng_ref,
                   bnb_ref, lnw_ref, lnb_ref, x_ref):
    exloop = exloop_ref[...]
    num = num_ref[...] + exloop * xl_ref[...]
    den = den_ref[...] + exloop + 1e-16
    h = num / den + gatb_ref[...]
    mu = jnp.mean(h, axis=0, keepdims=True)
    var = jnp.mean((h - mu) ** 2, axis=0, keepdims=True)
    h = (h - mu) * jax.lax.rsqrt(var + 1e-5) * bng_ref[...] + bnb_ref[...]
    h = jnp.maximum(h, 0.0)
    x_ref[...] = jnp.maximum(h @ lnw_ref[...] + lnb_ref[...], 0.0)


def _node_fin(num, den, exloop, xl, gatb, bng, bnb, lnw, lnb):
    n = num.shape[0]
    return pl.pallas_call(
        _node_fin_body,
        out_shape=jax.ShapeDtypeStruct((n, D), jnp.float32),
    )(num, den[:, None], exloop[:, None], xl, gatb[None, :], bng[None, :],
      bnb[None, :], lnw, lnb)


def _pool_mlp_body(x_ref, batch_ref, w1_ref, b1_ref, w2_ref, b2_ref,
                   w3_ref, b3_ref, out_ref):
    g = 16
    batch = batch_ref[...]
    onehot = (batch == jax.lax.broadcasted_iota(jnp.int32, (1, g), 1)
              ).astype(jnp.float32)
    sums = jnp.einsum('ng,nd->gd', onehot, x_ref[...],
                      preferred_element_type=jnp.float32)
    cnts = jnp.sum(onehot, axis=0)
    gm = sums / jnp.maximum(cnts, 1.0)[:, None]
    gm = jnp.maximum(gm @ w1_ref[...] + b1_ref[...], 0.0)
    gm = jnp.maximum(gm @ w2_ref[...] + b2_ref[...], 0.0)
    out_ref[...] = gm @ w3_ref[...] + b3_ref[...]


def _pool_mlp(x, batch, m):
    return pl.pallas_call(
        _pool_mlp_body,
        out_shape=jax.ShapeDtypeStruct((16, 1), jnp.float32),
    )(x, batch[:, None], m['W1'], m['b1'][None, :], m['W2'],
      m['b2'][None, :], m['W3'], m['b3'][None, :])


def kernel(x, edge_index, edge_attr, batch, params):
    n = x.shape[0]
    src = edge_index[0]
    dst = edge_index[1]
    ea = edge_attr
    for i in range(3):
        p = params['layer%d' % i]
        ein = ea.shape[1]
        w1 = p['emlp_W1']
        w1s, w1d, w1e = w1[:D], w1[D:2 * D], w1[2 * D:]
        ve = p['gat_W_edge'] @ p['gat_att_edge']
        xs1, xd1, xl, a_src, a_dst, exloop = _node_pre(
            x, w1s, w1d, p['emlp_b1'], p['gat_W'], p['gat_att_src'],
            p['gat_att_dst'])
        gs = xs1[src] + xd1[dst]
        sa = a_src[src] + a_dst[dst]
        ea, ex = _edge_dense(gs, ea, sa, w1e, p['emlp_W2'], p['emlp_b2'],
                             ve, p['le_W'], p['le_b'])
        num = jax.ops.segment_sum(ex[:, None] * xl[src], dst, num_segments=n)
        den = jax.ops.segment_sum(ex, dst, num_segments=n)
        x = _node_fin(num, den, exloop, xl, p['gat_b'], p['bn_g'],
                      p['bn_b'], p['ln_W'], p['ln_b'])
    return _pool_mlp(x, batch, params['mlp'])


# speedup vs baseline: 6.3865x; 4.0683x over previous
import functools

import jax
import jax.numpy as jnp
from jax import lax
from jax.experimental import pallas as pl
from jax.experimental.pallas import tpu as pltpu
from jax.experimental.pallas import tpu_sc as plsc

D = 128
LEAKY = 0.2
NC, NS, L = 2, 16, 16
NW = NC * NS
CHUNK = 128
E = 320000
NCHT = E // CHUNK
NN = 10000
NPW = 313
ACCR = 320
DW = D + 16


def _leaky(v):
    return jnp.where(v >= 0, v, LEAKY * v)


def _scalar_at(vec_ref, t):
    grp = t // L
    lane = t % L
    v = vec_ref[pl.ds(grp * L, L)]
    sel = jnp.where(lax.broadcasted_iota(jnp.int32, (L,), 0) == lane, v, 0)
    return jnp.max(sel, axis=0)


def _node_pre_body(x_ref, w1s_ref, w1d_ref, b1_ref, gatw_ref, asrc_ref,
                   adst_ref, xs1_ref, xd1_ref, xl_ref, av_ref):
    x = x_ref[...]
    xs1_ref[...] = x @ w1s_ref[...] + b1_ref[...]
    xd1_ref[...] = x @ w1d_ref[...]
    xl = x @ gatw_ref[...]
    xl_ref[...] = xl
    a_src = xl @ asrc_ref[...]
    a_dst = xl @ adst_ref[...]
    exloop = jnp.exp(_leaky(a_src + a_dst))
    av_ref[...] = jnp.concatenate([a_src, a_dst, exloop], axis=1)


def _node_pre(x, w1s, w1d, b1, gatw, att_src, att_dst):
    n = x.shape[0]
    out = pl.pallas_call(
        _node_pre_body,
        out_shape=[
            jax.ShapeDtypeStruct((n, D), jnp.float32),
            jax.ShapeDtypeStruct((n, D), jnp.float32),
            jax.ShapeDtypeStruct((n, D), jnp.float32),
            jax.ShapeDtypeStruct((n, 3), jnp.float32),
        ],
    )(x, w1s, w1d, b1[None, :], gatw, att_src[:, None], att_dst[:, None])
    xs1, xd1, xl, av = out
    return xs1, xd1, xl, av[:, 0], av[:, 1], av[:, 2:3]


def _sc_gather_body(xs1, xd1, asrc, adst, src2, dst2, gs, sa,
                    asrc_t, adst_t, idxs, idxd, rows_s, rows_d, sa_buf, sem):
    cid = lax.axis_index("c")
    sid = lax.axis_index("s")
    wid = cid * NS + sid
    c0 = wid * (NCHT // NW) + jnp.minimum(wid, NCHT % NW)
    nch = jnp.where(wid < NCHT % NW, NCHT // NW + 1, NCHT // NW)
    pltpu.sync_copy(asrc, asrc_t)
    pltpu.sync_copy(adst, adst_t)

    def chunk(j, carry):
        c = c0 + j
        pltpu.sync_copy(src2.at[c], idxs)
        pltpu.sync_copy(dst2.at[c], idxd)
        cp1 = pltpu.async_copy(xs1.at[idxs], rows_s, sem)
        cp2 = pltpu.async_copy(xd1.at[idxd], rows_d, sem)
        cp1.wait()
        cp2.wait()

        def rowadd(r, c2):
            for k in range(D // L):
                sl = pl.ds(k * L, L)
                rows_s[r, sl] = rows_s[r, sl] + rows_d[r, sl]
            return c2
        lax.fori_loop(0, CHUNK, rowadd, 0)
        for k in range(CHUNK // L):
            sl = pl.ds(k * L, L)
            sa_buf[sl] = (plsc.load_gather(asrc_t, [idxs[sl]])
                          + plsc.load_gather(adst_t, [idxd[sl]]))
        pltpu.sync_copy(rows_s, gs.at[pl.ds(c * CHUNK, CHUNK)])
        pltpu.sync_copy(sa_buf, sa.at[pl.ds(c * CHUNK, CHUNK)])
        return carry
    lax.fori_loop(0, nch, chunk, 0)


_sc_gather = pl.kernel(
    _sc_gather_body,
    out_type=[
        jax.ShapeDtypeStruct((E, D), jnp.float32),
        jax.ShapeDtypeStruct((E,), jnp.float32),
    ],
    mesh=plsc.VectorSubcoreMesh(core_axis_name="c", subcore_axis_name="s"),
    compiler_params=pltpu.CompilerParams(
        needs_layout_passes=False, use_tc_tiling_on_sc=False),
    scratch_types=[
        pltpu.VMEM((NN,), jnp.float32),
        pltpu.VMEM((NN,), jnp.float32),
        pltpu.VMEM((CHUNK,), jnp.int32),
        pltpu.VMEM((CHUNK,), jnp.int32),
        pltpu.VMEM((CHUNK, D), jnp.float32),
        pltpu.VMEM((CHUNK, D), jnp.float32),
        pltpu.VMEM((CHUNK,), jnp.float32),
        pltpu.SemaphoreType.DMA,
    ],
)


def _edge_dense_body(gs_ref, ea_ref, sa_ref, w1e_ref, w2_ref, b2_ref,
                     ve_ref, lew_ref, leb_ref, eaout_ref, ex_ref):
    ea = ea_ref[...]
    t = jnp.maximum(gs_ref[...] + ea @ w1e_ref[...], 0.0)
    h = t @ w2_ref[...] + b2_ref[...]
    ea_new = h + ea
    a_edge = ea_new @ ve_ref[...]
    alpha = _leaky(sa_ref[...] + a_edge)
    ex_ref[...] = jnp.exp(alpha)
    eaout_ref[...] = jnp.maximum(ea_new @ lew_ref[...] + leb_ref[...], 0.0)


def _edge_dense(gs, ea, sa, w1e, w2, b2, ve, lew, leb, blk=2000):
    e = gs.shape[0]
    ein = ea.shape[1]
    grid = (e // blk,)
    eaout, ex = pl.pallas_call(
        _edge_dense_body,
        grid=grid,
        in_specs=[
            pl.BlockSpec((blk, D), lambda i: (i, 0)),
            pl.BlockSpec((blk, ein), lambda i: (i, 0)),
            pl.BlockSpec((blk, 1), lambda i: (i, 0)),
            pl.BlockSpec((ein, D), lambda i: (0, 0)),
            pl.BlockSpec((D, D), lambda i: (0, 0)),
            pl.BlockSpec((1, D), lambda i: (0, 0)),
            pl.BlockSpec((D, 1), lambda i: (0, 0)),
            pl.BlockSpec((D, D), lambda i: (0, 0)),
            pl.BlockSpec((1, D), lambda i: (0, 0)),
        ],
        out_specs=[
            pl.BlockSpec((blk, D), lambda i: (i, 0)),
            pl.BlockSpec((blk, 1), lambda i: (i, 0)),
        ],
        out_shape=[
            jax.ShapeDtypeStruct((e, D), jnp.float32),
            jax.ShapeDtypeStruct((e, 1), jnp.float32),
        ],
    )(gs, ea, sa[:, None], w1e, w2, b2[None, :], ve[:, None], lew,
      leb[None, :])
    return eaout, ex[:, 0]


def _sc_scatter_body(xlp, exv, src2, dst2, offs, outp,
                     off_v, idxs, idxd, ex_buf, rows, acc, sem):
    cid = lax.axis_index("c")
    sid = lax.axis_index("s")
    wid = cid * NS + sid
    nstart = wid * NPW
    pltpu.sync_copy(offs, off_v)
    estart = _scalar_at(off_v, wid)
    eend = _scalar_at(off_v, wid + 1)
    ch0 = estart // CHUNK
    ch1 = (eend + CHUNK - 1) // CHUNK

    def zrow(r, carry):
        for k in range(DW // L):
            acc[r, pl.ds(k * L, L)] = jnp.zeros((L,), jnp.float32)
        return carry
    lax.fori_loop(0, ACCR, zrow, 0)

    iota = lax.broadcasted_iota(jnp.int32, (L,), 0)

    def chunk(c, carry):
        pltpu.sync_copy(src2.at[c], idxs)
        pltpu.sync_copy(dst2.at[c], idxd)
        pltpu.sync_copy(exv.at[pl.ds(c * CHUNK, CHUNK)], ex_buf)
        pltpu.async_copy(xlp.at[idxs], rows, sem).wait()

        def rowacc(r, c2):
            e_glob = c * CHUNK + r
            valid = jnp.logical_and(e_glob >= estart, e_glob < eend)
            rsplat = jnp.zeros((L,), jnp.int32) + r
            ev = plsc.load_gather(ex_buf, [rsplat])
            ev = jnp.where(valid, ev, 0.0)
            dd = plsc.load_gather(idxd, [rsplat]) - nstart
            dd = jnp.where(valid, dd, ACCR - 1)
            for k in range(DW // L):
                sl = pl.ds(k * L, L)
                plsc.addupdate_scatter(
                    acc, [dd, iota + (k * L)], rows[r, sl] * ev)
            return c2
        lax.fori_loop(0, CHUNK, rowacc, 0)
        return carry
    lax.fori_loop(ch0, ch1, chunk, 0)
    pltpu.sync_copy(acc.at[pl.ds(0, NPW)], outp.at[pl.ds(nstart, NPW)])


_sc_scatter = pl.kernel(
    _sc_scatter_body,
    out_type=jax.ShapeDtypeStruct((NW * NPW, DW), jnp.float32),
    mesh=plsc.VectorSubcoreMesh(core_axis_name="c", subcore_axis_name="s"),
    compiler_params=pltpu.CompilerParams(
        needs_layout_passes=False, use_tc_tiling_on_sc=False),
    scratch_types=[
        pltpu.VMEM((3 * L,), jnp.int32),
        pltpu.VMEM((CHUNK,), jnp.int32),
        pltpu.VMEM((CHUNK,), jnp.int32),
        pltpu.VMEM((CHUNK,), jnp.float32),
        pltpu.VMEM((CHUNK, DW), jnp.float32),
        pltpu.VMEM((ACCR, DW), jnp.float32),
        pltpu.SemaphoreType.DMA,
    ],
)


def _node_fin_body(acc_ref, exloop_ref, xl_ref, gatb_ref, bng_ref,
                   bnb_ref, lnw_ref, lnb_ref, x_ref):
    exloop = exloop_ref[...]
    xl = xl_ref[...]
    a = acc_ref[...]
    num = a[:, :D] + exloop * xl
    den = jnp.sum(a[:, D:], axis=1, keepdims=True) + exloop + 1e-16
    h = num / den + gatb_ref[...]
    mu = jnp.mean(h, axis=0, keepdims=True)
    var = jnp.mean((h - mu) ** 2, axis=0, keepdims=True)
    h = (h - mu) * jax.lax.rsqrt(var + 1e-5) * bng_ref[...] + bnb_ref[...]
    h = jnp.maximum(h, 0.0)
    x_ref[...] = jnp.maximum(h @ lnw_ref[...] + lnb_ref[...], 0.0)


def _node_fin(acc, exloop, xl, gatb, bng, bnb, lnw, lnb):
    n = xl.shape[0]
    return pl.pallas_call(
        _node_fin_body,
        out_shape=jax.ShapeDtypeStruct((n, D), jnp.float32),
    )(acc, exloop, xl, gatb[None, :], bng[None, :],
      bnb[None, :], lnw, lnb)


def _pool_mlp_body(x_ref, batch_ref, w1_ref, b1_ref, w2_ref, b2_ref,
                   w3_ref, b3_ref, out_ref):
    g = 16
    batch = batch_ref[...]
    onehot = (batch == jax.lax.broadcasted_iota(jnp.int32, (1, g), 1)
              ).astype(jnp.float32)
    sums = jnp.einsum('ng,nd->gd', onehot, x_ref[...],
                      preferred_element_type=jnp.float32)
    cnts = jnp.sum(onehot, axis=0)
    gm = sums / jnp.maximum(cnts, 1.0)[:, None]
    gm = jnp.maximum(gm @ w1_ref[...] + b1_ref[...], 0.0)
    gm = jnp.maximum(gm @ w2_ref[...] + b2_ref[...], 0.0)
    out_ref[...] = gm @ w3_ref[...] + b3_ref[...]


def _pool_mlp(x, batch, m):
    return pl.pallas_call(
        _pool_mlp_body,
        out_shape=jax.ShapeDtypeStruct((16, 1), jnp.float32),
    )(x, batch[:, None], m['W1'], m['b1'][None, :], m['W2'],
      m['b2'][None, :], m['W3'], m['b3'][None, :])


def kernel(x, edge_index, edge_attr, batch, params):
    n = x.shape[0]
    order = jnp.argsort(edge_index[1])
    src = edge_index[0][order]
    dst = edge_index[1][order]
    src2 = src.reshape(NCHT, CHUNK)
    dst2 = dst.reshape(NCHT, CHUNK)
    ea = edge_attr[order]
    nb = jnp.arange(NW + 1, dtype=jnp.int32) * NPW
    offs = jnp.pad(jnp.searchsorted(dst, nb).astype(jnp.int32),
                   (0, 3 * L - (NW + 1)))
    ones16 = jnp.concatenate(
        [jnp.ones((n, 1), jnp.float32), jnp.zeros((n, 15), jnp.float32)], 1)
    for i in range(3):
        p = params['layer%d' % i]
        w1 = p['emlp_W1']
        w1s, w1d, w1e = w1[:D], w1[D:2 * D], w1[2 * D:]
        ve = p['gat_W_edge'] @ p['gat_att_edge']
        xs1, xd1, xl, a_src, a_dst, exloop = _node_pre(
            x, w1s, w1d, p['emlp_b1'], p['gat_W'], p['gat_att_src'],
            p['gat_att_dst'])
        gs, sa = _sc_gather(xs1, xd1, a_src, a_dst, src2, dst2)
        ea, ex = _edge_dense(gs, ea, sa, w1e, p['emlp_W2'], p['emlp_b2'],
                             ve, p['le_W'], p['le_b'])
        xlp = jnp.concatenate([xl, ones16], axis=1)
        acc = _sc_scatter(xlp, ex, src2, dst2, offs)
        x = _node_fin(acc[:n], exloop, xl, p['gat_b'],
                      p['bn_g'], p['bn_b'], p['ln_W'], p['ln_b'])
    return _pool_mlp(x, batch, params['mlp'])


# speedup vs baseline: 7.0079x; 1.0973x over previous
import functools

import jax
import jax.numpy as jnp
from jax import lax
from jax.experimental import pallas as pl
from jax.experimental.pallas import tpu as pltpu
from jax.experimental.pallas import tpu_sc as plsc

D = 128
LEAKY = 0.2
NC, NS, L = 2, 16, 16
NW = NC * NS
CHUNK = 128
E = 320000
NCHT = E // CHUNK
NN = 10000
NPW = 313
ACCR = 320
DW = D + 16


def _leaky(v):
    return jnp.where(v >= 0, v, LEAKY * v)


def _scalar_at(vec_ref, t):
    grp = t // L
    lane = t % L
    v = vec_ref[pl.ds(grp * L, L)]
    sel = jnp.where(lax.broadcasted_iota(jnp.int32, (L,), 0) == lane, v, 0)
    return jnp.max(sel, axis=0)


def _node_pre_body(x_ref, w1s_ref, w1d_ref, b1_ref, gatw_ref, asrc_ref,
                   adst_ref, xs1_ref, xd1_ref, xl_ref, av_ref):
    x = x_ref[...]
    xs1_ref[...] = x @ w1s_ref[...] + b1_ref[...]
    xd1_ref[...] = x @ w1d_ref[...]
    xl = x @ gatw_ref[...]
    xl_ref[...] = xl
    a_src = xl @ asrc_ref[...]
    a_dst = xl @ adst_ref[...]
    exloop = jnp.exp(_leaky(a_src + a_dst))
    av_ref[...] = jnp.concatenate([a_src, a_dst, exloop], axis=1)


def _node_pre(x, w1s, w1d, b1, gatw, att_src, att_dst):
    n = x.shape[0]
    out = pl.pallas_call(
        _node_pre_body,
        out_shape=[
            jax.ShapeDtypeStruct((n, D), jnp.float32),
            jax.ShapeDtypeStruct((n, D), jnp.float32),
            jax.ShapeDtypeStruct((n, D), jnp.float32),
            jax.ShapeDtypeStruct((n, 3), jnp.float32),
        ],
    )(x, w1s, w1d, b1[None, :], gatw, att_src[:, None], att_dst[:, None])
    xs1, xd1, xl, av = out
    return xs1, xd1, xl, av[:, 0], av[:, 1], av[:, 2:3]


def _sc_gather_body(xs1, xd1, asrc, adst, src2, dst2, gs, sa,
                    asrc_t, adst_t, idxs0, idxd0, idxs1, idxd1,
                    rows_s0, rows_d0, rows_s1, rows_d1, sa_buf,
                    sem, sem_i):
    cid = lax.axis_index("c")
    sid = lax.axis_index("s")
    wid = cid * NS + sid
    c0 = wid * (NCHT // NW) + jnp.minimum(wid, NCHT % NW)
    nch = jnp.where(wid < NCHT % NW, NCHT // NW + 1, NCHT // NW)
    pltpu.sync_copy(asrc, asrc_t)
    pltpu.sync_copy(adst, adst_t)

    def fire_idx(j, i_s, i_d):
        pltpu.async_copy(src2.at[c0 + j], i_s, sem_i)
        pltpu.async_copy(dst2.at[c0 + j], i_d, sem_i)

    def wait_idx(i_s, i_d):
        pltpu.make_async_copy(src2.at[c0], i_s, sem_i).wait()
        pltpu.make_async_copy(dst2.at[c0], i_d, sem_i).wait()

    def fire_rows(i_s, i_d, r_s, r_d):
        pltpu.async_copy(xs1.at[i_s], r_s, sem)
        pltpu.async_copy(xd1.at[i_d], r_d, sem)

    def wait_rows(i_s, i_d, r_s, r_d):
        pltpu.make_async_copy(xs1.at[i_s], r_s, sem).wait()
        pltpu.make_async_copy(xd1.at[i_d], r_d, sem).wait()

    def process(j, i_s, i_d, r_s, r_d):
        def rowadd(r, c2):
            for k in range(D // L):
                sl = pl.ds(k * L, L)
                r_s[r, sl] = r_s[r, sl] + r_d[r, sl]
            return c2
        lax.fori_loop(0, CHUNK, rowadd, 0)
        for k in range(CHUNK // L):
            sl = pl.ds(k * L, L)
            sa_buf[sl] = (plsc.load_gather(asrc_t, [i_s[sl]])
                          + plsc.load_gather(adst_t, [i_d[sl]]))
        c = c0 + j
        pltpu.sync_copy(r_s, gs.at[pl.ds(c * CHUNK, CHUNK)])
        pltpu.sync_copy(sa_buf, sa.at[pl.ds(c * CHUNK, CHUNK)])

    bufs = ((idxs0, idxd0, rows_s0, rows_d0),
            (idxs1, idxd1, rows_s1, rows_d1))

    def half(j, cur, nxt):
        @pl.when(j < nch)
        def _():
            wait_rows(*cur)

        @pl.when(j + 1 < nch)
        def _():
            fire_idx(j + 1, nxt[0], nxt[1])

        @pl.when(j < nch)
        def _():
            process(j, *cur)

        @pl.when(j + 1 < nch)
        def _():
            wait_idx(nxt[0], nxt[1])
            fire_rows(*nxt)

    fire_idx(0, idxs0, idxd0)
    wait_idx(idxs0, idxd0)
    fire_rows(*bufs[0])

    def pair(t, carry):
        half(2 * t, bufs[0], bufs[1])
        half(2 * t + 1, bufs[1], bufs[0])
        return carry
    lax.fori_loop(0, (nch + 1) // 2, pair, 0)


_sc_gather = pl.kernel(
    _sc_gather_body,
    out_type=[
        jax.ShapeDtypeStruct((E, D), jnp.float32),
        jax.ShapeDtypeStruct((E,), jnp.float32),
    ],
    mesh=plsc.VectorSubcoreMesh(core_axis_name="c", subcore_axis_name="s"),
    compiler_params=pltpu.CompilerParams(
        needs_layout_passes=False, use_tc_tiling_on_sc=False),
    scratch_types=[
        pltpu.VMEM((NN,), jnp.float32),
        pltpu.VMEM((NN,), jnp.float32),
        pltpu.VMEM((CHUNK,), jnp.int32),
        pltpu.VMEM((CHUNK,), jnp.int32),
        pltpu.VMEM((CHUNK,), jnp.int32),
        pltpu.VMEM((CHUNK,), jnp.int32),
        pltpu.VMEM((CHUNK, D), jnp.float32),
        pltpu.VMEM((CHUNK, D), jnp.float32),
        pltpu.VMEM((CHUNK, D), jnp.float32),
        pltpu.VMEM((CHUNK, D), jnp.float32),
        pltpu.VMEM((CHUNK,), jnp.float32),
        pltpu.SemaphoreType.DMA,
        pltpu.SemaphoreType.DMA,
    ],
)


def _edge_dense_body(gs_ref, ea_ref, sa_ref, w1e_ref, w2_ref, b2_ref,
                     ve_ref, lew_ref, leb_ref, eaout_ref, ex_ref):
    ea = ea_ref[...]
    t = jnp.maximum(gs_ref[...] + ea @ w1e_ref[...], 0.0)
    h = t @ w2_ref[...] + b2_ref[...]
    ea_new = h + ea
    a_edge = ea_new @ ve_ref[...]
    alpha = _leaky(sa_ref[...] + a_edge)
    ex_ref[...] = jnp.exp(alpha)
    eaout_ref[...] = jnp.maximum(ea_new @ lew_ref[...] + leb_ref[...], 0.0)


def _edge_dense(gs, ea, sa, w1e, w2, b2, ve, lew, leb, blk=2000):
    e = gs.shape[0]
    ein = ea.shape[1]
    grid = (e // blk,)
    eaout, ex = pl.pallas_call(
        _edge_dense_body,
        grid=grid,
        in_specs=[
            pl.BlockSpec((blk, D), lambda i: (i, 0)),
            pl.BlockSpec((blk, ein), lambda i: (i, 0)),
            pl.BlockSpec((blk, 1), lambda i: (i, 0)),
            pl.BlockSpec((ein, D), lambda i: (0, 0)),
            pl.BlockSpec((D, D), lambda i: (0, 0)),
            pl.BlockSpec((1, D), lambda i: (0, 0)),
            pl.BlockSpec((D, 1), lambda i: (0, 0)),
            pl.BlockSpec((D, D), lambda i: (0, 0)),
            pl.BlockSpec((1, D), lambda i: (0, 0)),
        ],
        out_specs=[
            pl.BlockSpec((blk, D), lambda i: (i, 0)),
            pl.BlockSpec((blk, 1), lambda i: (i, 0)),
        ],
        out_shape=[
            jax.ShapeDtypeStruct((e, D), jnp.float32),
            jax.ShapeDtypeStruct((e, 1), jnp.float32),
        ],
    )(gs, ea, sa[:, None], w1e, w2, b2[None, :], ve[:, None], lew,
      leb[None, :])
    return eaout, ex[:, 0]


def _sc_scatter_body(xlp, exv, src2, dst2, offs, outp,
                     off_v, idxs0, idxd0, ex0, idxs1, idxd1, ex1,
                     rows0, rows1, acc, sem, sem_i):
    cid = lax.axis_index("c")
    sid = lax.axis_index("s")
    wid = cid * NS + sid
    nstart = wid * NPW
    pltpu.sync_copy(offs, off_v)
    estart = _scalar_at(off_v, wid)
    eend = _scalar_at(off_v, wid + 1)
    ch0 = estart // CHUNK
    ch1 = (eend + CHUNK - 1) // CHUNK
    nchf = ch1 - ch0

    iota = lax.broadcasted_iota(jnp.int32, (L,), 0)

    def fire_idx(j, i_s, i_d, i_e):
        c = ch0 + j
        pltpu.async_copy(src2.at[c], i_s, sem_i)
        pltpu.async_copy(dst2.at[c], i_d, sem_i)
        pltpu.async_copy(exv.at[pl.ds(c * CHUNK, CHUNK)], i_e, sem_i)

    def wait_idx(i_s, i_d, i_e):
        pltpu.make_async_copy(src2.at[ch0], i_s, sem_i).wait()
        pltpu.make_async_copy(dst2.at[ch0], i_d, sem_i).wait()
        pltpu.make_async_copy(exv.at[pl.ds(0, CHUNK)], i_e, sem_i).wait()

    def process(j, i_s, i_d, i_e, r_b):
        c = ch0 + j

        def rowacc(r, c2):
            e_glob = c * CHUNK + r
            valid = jnp.logical_and(e_glob >= estart, e_glob < eend)
            rsplat = jnp.zeros((L,), jnp.int32) + r
            ev = plsc.load_gather(i_e, [rsplat])
            ev = jnp.where(valid, ev, 0.0)
            dd = plsc.load_gather(i_d, [rsplat]) - nstart
            dd = jnp.where(valid, dd, ACCR - 1)
            for k in range(DW // L):
                sl = pl.ds(k * L, L)
                plsc.addupdate_scatter(
                    acc, [dd, iota + (k * L)], r_b[r, sl] * ev)
            return c2
        lax.fori_loop(0, CHUNK, rowacc, 0)

    def zrow(r, carry):
        for k in range(DW // L):
            acc[r, pl.ds(k * L, L)] = jnp.zeros((L,), jnp.float32)
        return carry
    lax.fori_loop(0, ACCR, zrow, 0)

    bufs = ((idxs0, idxd0, ex0, rows0), (idxs1, idxd1, ex1, rows1))

    def half(j, cur, nxt):
        @pl.when(j < nchf)
        def _():
            pltpu.make_async_copy(xlp.at[cur[0]], cur[3], sem).wait()

        @pl.when(j + 1 < nchf)
        def _():
            fire_idx(j + 1, nxt[0], nxt[1], nxt[2])

        @pl.when(j < nchf)
        def _():
            process(j, *cur)

        @pl.when(j + 1 < nchf)
        def _():
            wait_idx(nxt[0], nxt[1], nxt[2])
            pltpu.async_copy(xlp.at[nxt[0]], nxt[3], sem)

    @pl.when(nchf > 0)
    def _():
        fire_idx(0, idxs0, idxd0, ex0)
        wait_idx(idxs0, idxd0, ex0)
        pltpu.async_copy(xlp.at[idxs0], rows0, sem)

    def pair(t, carry):
        half(2 * t, bufs[0], bufs[1])
        half(2 * t + 1, bufs[1], bufs[0])
        return carry
    lax.fori_loop(0, (nchf + 1) // 2, pair, 0)
    pltpu.sync_copy(acc.at[pl.ds(0, NPW)], outp.at[pl.ds(nstart, NPW)])


_sc_scatter = pl.kernel(
    _sc_scatter_body,
    out_type=jax.ShapeDtypeStruct((NW * NPW, DW), jnp.float32),
    mesh=plsc.VectorSubcoreMesh(core_axis_name="c", subcore_axis_name="s"),
    compiler_params=pltpu.CompilerParams(
        needs_layout_passes=False, use_tc_tiling_on_sc=False),
    scratch_types=[
        pltpu.VMEM((3 * L,), jnp.int32),
        pltpu.VMEM((CHUNK,), jnp.int32),
        pltpu.VMEM((CHUNK,), jnp.int32),
        pltpu.VMEM((CHUNK,), jnp.float32),
        pltpu.VMEM((CHUNK,), jnp.int32),
        pltpu.VMEM((CHUNK,), jnp.int32),
        pltpu.VMEM((CHUNK,), jnp.float32),
        pltpu.VMEM((CHUNK, DW), jnp.float32),
        pltpu.VMEM((CHUNK, DW), jnp.float32),
        pltpu.VMEM((ACCR, DW), jnp.float32),
        pltpu.SemaphoreType.DMA,
        pltpu.SemaphoreType.DMA,
    ],
)


def _node_fin_body(acc_ref, exloop_ref, xl_ref, gatb_ref, bng_ref,
                   bnb_ref, lnw_ref, lnb_ref, x_ref):
    exloop = exloop_ref[...]
    xl = xl_ref[...]
    a = acc_ref[...]
    num = a[:, :D] + exloop * xl
    den = jnp.sum(a[:, D:], axis=1, keepdims=True) + exloop + 1e-16
    h = num / den + gatb_ref[...]
    mu = jnp.mean(h, axis=0, keepdims=True)
    var = jnp.mean((h - mu) ** 2, axis=0, keepdims=True)
    h = (h - mu) * jax.lax.rsqrt(var + 1e-5) * bng_ref[...] + bnb_ref[...]
    h = jnp.maximum(h, 0.0)
    x_ref[...] = jnp.maximum(h @ lnw_ref[...] + lnb_ref[...], 0.0)


def _node_fin(acc, exloop, xl, gatb, bng, bnb, lnw, lnb):
    n = xl.shape[0]
    return pl.pallas_call(
        _node_fin_body,
        out_shape=jax.ShapeDtypeStruct((n, D), jnp.float32),
    )(acc, exloop, xl, gatb[None, :], bng[None, :],
      bnb[None, :], lnw, lnb)


def _pool_mlp_body(x_ref, batch_ref, w1_ref, b1_ref, w2_ref, b2_ref,
                   w3_ref, b3_ref, out_ref):
    g = 16
    batch = batch_ref[...]
    onehot = (batch == jax.lax.broadcasted_iota(jnp.int32, (1, g), 1)
              ).astype(jnp.float32)
    sums = jnp.einsum('ng,nd->gd', onehot, x_ref[...],
                      preferred_element_type=jnp.float32)
    cnts = jnp.sum(onehot, axis=0)
    gm = sums / jnp.maximum(cnts, 1.0)[:, None]
    gm = jnp.maximum(gm @ w1_ref[...] + b1_ref[...], 0.0)
    gm = jnp.maximum(gm @ w2_ref[...] + b2_ref[...], 0.0)
    out_ref[...] = gm @ w3_ref[...] + b3_ref[...]


def _pool_mlp(x, batch, m):
    return pl.pallas_call(
        _pool_mlp_body,
        out_shape=jax.ShapeDtypeStruct((16, 1), jnp.float32),
    )(x, batch[:, None], m['W1'], m['b1'][None, :], m['W2'],
      m['b2'][None, :], m['W3'], m['b3'][None, :])


def kernel(x, edge_index, edge_attr, batch, params):
    n = x.shape[0]
    order = jnp.argsort(edge_index[1])
    src = edge_index[0][order]
    dst = edge_index[1][order]
    src2 = src.reshape(NCHT, CHUNK)
    dst2 = dst.reshape(NCHT, CHUNK)
    ea = edge_attr[order]
    nb = jnp.arange(NW + 1, dtype=jnp.int32) * NPW
    offs = jnp.pad(jnp.searchsorted(dst, nb).astype(jnp.int32),
                   (0, 3 * L - (NW + 1)))
    ones16 = jnp.concatenate(
        [jnp.ones((n, 1), jnp.float32), jnp.zeros((n, 15), jnp.float32)], 1)
    for i in range(3):
        p = params['layer%d' % i]
        w1 = p['emlp_W1']
        w1s, w1d, w1e = w1[:D], w1[D:2 * D], w1[2 * D:]
        ve = p['gat_W_edge'] @ p['gat_att_edge']
        xs1, xd1, xl, a_src, a_dst, exloop = _node_pre(
            x, w1s, w1d, p['emlp_b1'], p['gat_W'], p['gat_att_src'],
            p['gat_att_dst'])
        gs, sa = _sc_gather(xs1, xd1, a_src, a_dst, src2, dst2)
        ea, ex = _edge_dense(gs, ea, sa, w1e, p['emlp_W2'], p['emlp_b2'],
                             ve, p['le_W'], p['le_b'])
        xlp = jnp.concatenate([xl, ones16], axis=1)
        acc = _sc_scatter(xlp, ex, src2, dst2, offs)
        x = _node_fin(acc[:n], exloop, xl, p['gat_b'],
                      p['bn_g'], p['bn_b'], p['ln_W'], p['ln_b'])
    return _pool_mlp(x, batch, params['mlp'])


# speedup vs baseline: 7.8957x; 1.1267x over previous
import functools

import jax
import jax.numpy as jnp
from jax import lax
from jax.experimental import pallas as pl
from jax.experimental.pallas import tpu as pltpu
from jax.experimental.pallas import tpu_sc as plsc

D = 128
LEAKY = 0.2
NC, NS, L = 2, 16, 16
NW = NC * NS
CHUNK = 128
E = 320000
NCHT = E // CHUNK
NN = 10000
NPW = 313
ACCR = 320
DW = D + 16


def _leaky(v):
    return jnp.where(v >= 0, v, LEAKY * v)


def _scalar_at(vec_ref, t):
    grp = t // L
    lane = t % L
    v = vec_ref[pl.ds(grp * L, L)]
    sel = jnp.where(lax.broadcasted_iota(jnp.int32, (L,), 0) == lane, v, 0)
    return jnp.max(sel, axis=0)


def _node_pre_body(x_ref, w1s_ref, w1d_ref, b1_ref, gatw_ref, asrc_ref,
                   adst_ref, xs1_ref, xd1_ref, xl_ref, av_ref):
    x = x_ref[...]
    xs1_ref[...] = x @ w1s_ref[...] + b1_ref[...]
    xd1_ref[...] = x @ w1d_ref[...]
    xl = x @ gatw_ref[...]
    xl_ref[...] = xl
    a_src = xl @ asrc_ref[...]
    a_dst = xl @ adst_ref[...]
    exloop = jnp.exp(_leaky(a_src + a_dst))
    av_ref[...] = jnp.concatenate([a_src, a_dst, exloop], axis=1)


def _node_pre(x, w1s, w1d, b1, gatw, att_src, att_dst):
    n = x.shape[0]
    out = pl.pallas_call(
        _node_pre_body,
        out_shape=[
            jax.ShapeDtypeStruct((n, D), jnp.float32),
            jax.ShapeDtypeStruct((n, D), jnp.float32),
            jax.ShapeDtypeStruct((n, D), jnp.float32),
            jax.ShapeDtypeStruct((n, 3), jnp.float32),
        ],
    )(x, w1s, w1d, b1[None, :], gatw, att_src[:, None], att_dst[:, None])
    xs1, xd1, xl, av = out
    return xs1, xd1, xl, av[:, 0], av[:, 1], av[:, 2:3]


def _sc_gather_body(xs1, xd1, asrc, adst, src2, dst2, gs, sa,
                    asrc_t, adst_t, idxs0, idxd0, idxs1, idxd1,
                    rows_s0, rows_d0, rows_s1, rows_d1, sa_buf,
                    sem, sem_i):
    cid = lax.axis_index("c")
    sid = lax.axis_index("s")
    wid = cid * NS + sid
    c0 = wid * (NCHT // NW) + jnp.minimum(wid, NCHT % NW)
    nch = jnp.where(wid < NCHT % NW, NCHT // NW + 1, NCHT // NW)
    pltpu.sync_copy(asrc, asrc_t)
    pltpu.sync_copy(adst, adst_t)

    def fire_idx(j, i_s, i_d):
        pltpu.async_copy(src2.at[c0 + j], i_s, sem_i)
        pltpu.async_copy(dst2.at[c0 + j], i_d, sem_i)

    def wait_idx(i_s, i_d):
        pltpu.make_async_copy(src2.at[c0], i_s, sem_i).wait()
        pltpu.make_async_copy(dst2.at[c0], i_d, sem_i).wait()

    def fire_rows(i_s, i_d, r_s, r_d):
        pltpu.async_copy(xs1.at[i_s], r_s, sem)
        pltpu.async_copy(xd1.at[i_d], r_d, sem)

    def wait_rows(i_s, i_d, r_s, r_d):
        pltpu.make_async_copy(xs1.at[i_s], r_s, sem).wait()
        pltpu.make_async_copy(xd1.at[i_d], r_d, sem).wait()

    def process(j, i_s, i_d, r_s, r_d):
        del i_s, i_d

        def rowadd(r, c2):
            for k in range(D // L):
                sl = pl.ds(k * L, L)
                r_s[r, sl] = r_s[r, sl] + r_d[r, sl]
            return c2
        lax.fori_loop(0, CHUNK, rowadd, 0)
        c = c0 + j
        pltpu.sync_copy(r_s, gs.at[pl.ds(c * CHUNK, CHUNK)])
        pltpu.sync_copy(sa_buf, sa.at[pl.ds(c * CHUNK, CHUNK)])

    bufs = ((idxs0, idxd0, rows_s0, rows_d0),
            (idxs1, idxd1, rows_s1, rows_d1))

    def half(j, cur, nxt):
        @pl.when(j < nch)
        def _():
            wait_rows(*cur)
            for k in range(CHUNK // L):
                sl = pl.ds(k * L, L)
                sa_buf[sl] = (plsc.load_gather(asrc_t, [cur[0][sl]])
                              + plsc.load_gather(adst_t, [cur[1][sl]]))

        @pl.when(j + 1 < nch)
        def _():
            wait_idx(nxt[0], nxt[1])
            fire_rows(*nxt)

        @pl.when(j + 2 < nch)
        def _():
            fire_idx(j + 2, cur[0], cur[1])

        @pl.when(j < nch)
        def _():
            process(j, *cur)

    fire_idx(0, idxs0, idxd0)
    wait_idx(idxs0, idxd0)
    fire_rows(*bufs[0])
    fire_idx(1, idxs1, idxd1)

    def pair(t, carry):
        half(2 * t, bufs[0], bufs[1])
        half(2 * t + 1, bufs[1], bufs[0])
        return carry
    lax.fori_loop(0, (nch + 1) // 2, pair, 0)


_sc_gather = pl.kernel(
    _sc_gather_body,
    out_type=[
        jax.ShapeDtypeStruct((E, D), jnp.float32),
        jax.ShapeDtypeStruct((E,), jnp.float32),
    ],
    mesh=plsc.VectorSubcoreMesh(core_axis_name="c", subcore_axis_name="s"),
    compiler_params=pltpu.CompilerParams(
        needs_layout_passes=False, use_tc_tiling_on_sc=False),
    scratch_types=[
        pltpu.VMEM((NN,), jnp.float32),
        pltpu.VMEM((NN,), jnp.float32),
        pltpu.VMEM((CHUNK,), jnp.int32),
        pltpu.VMEM((CHUNK,), jnp.int32),
        pltpu.VMEM((CHUNK,), jnp.int32),
        pltpu.VMEM((CHUNK,), jnp.int32),
        pltpu.VMEM((CHUNK, D), jnp.float32),
        pltpu.VMEM((CHUNK, D), jnp.float32),
        pltpu.VMEM((CHUNK, D), jnp.float32),
        pltpu.VMEM((CHUNK, D), jnp.float32),
        pltpu.VMEM((CHUNK,), jnp.float32),
        pltpu.SemaphoreType.DMA,
        pltpu.SemaphoreType.DMA,
    ],
)


def _edge_dense_body(gs_ref, ea_ref, sa_ref, w1e_ref, w2_ref, b2_ref,
                     ve_ref, lew_ref, leb_ref, eaout_ref, ex_ref):
    ea = ea_ref[...]
    t = jnp.maximum(gs_ref[...] + ea @ w1e_ref[...], 0.0)
    h = t @ w2_ref[...] + b2_ref[...]
    ea_new = h + ea
    a_edge = ea_new @ ve_ref[...]
    alpha = _leaky(sa_ref[...] + a_edge)
    ex_ref[...] = jnp.exp(alpha)
    eaout_ref[...] = jnp.maximum(ea_new @ lew_ref[...] + leb_ref[...], 0.0)


def _edge_dense(gs, ea, sa, w1e, w2, b2, ve, lew, leb, blk=2000):
    e = gs.shape[0]
    ein = ea.shape[1]
    grid = (e // blk,)
    eaout, ex = pl.pallas_call(
        _edge_dense_body,
        grid=grid,
        in_specs=[
            pl.BlockSpec((blk, D), lambda i: (i, 0)),
            pl.BlockSpec((blk, ein), lambda i: (i, 0)),
            pl.BlockSpec((blk, 1), lambda i: (i, 0)),
            pl.BlockSpec((ein, D), lambda i: (0, 0)),
            pl.BlockSpec((D, D), lambda i: (0, 0)),
            pl.BlockSpec((1, D), lambda i: (0, 0)),
            pl.BlockSpec((D, 1), lambda i: (0, 0)),
            pl.BlockSpec((D, D), lambda i: (0, 0)),
            pl.BlockSpec((1, D), lambda i: (0, 0)),
        ],
        out_specs=[
            pl.BlockSpec((blk, D), lambda i: (i, 0)),
            pl.BlockSpec((blk, 1), lambda i: (i, 0)),
        ],
        out_shape=[
            jax.ShapeDtypeStruct((e, D), jnp.float32),
            jax.ShapeDtypeStruct((e, 1), jnp.float32),
        ],
    )(gs, ea, sa[:, None], w1e, w2, b2[None, :], ve[:, None], lew,
      leb[None, :])
    return eaout, ex[:, 0]


def _sc_scatter_body(xlp, exv, src2, dst2, offs, outp,
                     off_v, idxs0, idxd0, ex0, idxs1, idxd1, ex1,
                     rows0, rows1, ev_buf, dd_buf, acc, sem, sem_i):
    cid = lax.axis_index("c")
    sid = lax.axis_index("s")
    wid = cid * NS + sid
    nstart = wid * NPW
    pltpu.sync_copy(offs, off_v)
    estart = _scalar_at(off_v, wid)
    eend = _scalar_at(off_v, wid + 1)
    ch0 = estart // CHUNK
    ch1 = (eend + CHUNK - 1) // CHUNK
    nchf = ch1 - ch0

    iota = lax.broadcasted_iota(jnp.int32, (L,), 0)

    def fire_idx(j, i_s, i_d, i_e):
        c = ch0 + j
        pltpu.async_copy(src2.at[c], i_s, sem_i)
        pltpu.async_copy(dst2.at[c], i_d, sem_i)
        pltpu.async_copy(exv.at[pl.ds(c * CHUNK, CHUNK)], i_e, sem_i)

    def wait_idx(i_s, i_d, i_e):
        pltpu.make_async_copy(src2.at[ch0], i_s, sem_i).wait()
        pltpu.make_async_copy(dst2.at[ch0], i_d, sem_i).wait()
        pltpu.make_async_copy(exv.at[pl.ds(0, CHUNK)], i_e, sem_i).wait()

    def preproc(j, i_d, i_e):
        c = ch0 + j
        for k in range(CHUNK // L):
            sl = pl.ds(k * L, L)
            e_glob = iota + (c * CHUNK + k * L)
            valid = jnp.logical_and(e_glob >= estart, e_glob < eend)
            ev_buf[sl] = jnp.where(valid, i_e[sl], 0.0)
            dd_buf[sl] = jnp.where(valid, i_d[sl] - nstart, ACCR - 1)

    def process(r_b):
        def rowacc(r, c2):
            rsplat = jnp.zeros((L,), jnp.int32) + r
            ev = plsc.load_gather(ev_buf, [rsplat])
            dd = plsc.load_gather(dd_buf, [rsplat])
            for k in range(DW // L):
                sl = pl.ds(k * L, L)
                plsc.addupdate_scatter(
                    acc, [dd, iota + (k * L)], r_b[r, sl] * ev)
            return c2
        lax.fori_loop(0, CHUNK, rowacc, 0)

    def zrow(r, carry):
        for k in range(DW // L):
            acc[r, pl.ds(k * L, L)] = jnp.zeros((L,), jnp.float32)
        return carry
    lax.fori_loop(0, ACCR, zrow, 0)

    bufs = ((idxs0, idxd0, ex0, rows0), (idxs1, idxd1, ex1, rows1))

    def half(j, cur, nxt):
        @pl.when(j < nchf)
        def _():
            pltpu.make_async_copy(xlp.at[cur[0]], cur[3], sem).wait()
            preproc(j, cur[1], cur[2])

        @pl.when(j + 1 < nchf)
        def _():
            wait_idx(nxt[0], nxt[1], nxt[2])
            pltpu.async_copy(xlp.at[nxt[0]], nxt[3], sem)

        @pl.when(j + 2 < nchf)
        def _():
            fire_idx(j + 2, cur[0], cur[1], cur[2])

        @pl.when(j < nchf)
        def _():
            process(cur[3])

    @pl.when(nchf > 0)
    def _():
        fire_idx(0, idxs0, idxd0, ex0)
        wait_idx(idxs0, idxd0, ex0)
        pltpu.async_copy(xlp.at[idxs0], rows0, sem)

    @pl.when(nchf > 1)
    def _():
        fire_idx(1, idxs1, idxd1, ex1)

    def pair(t, carry):
        half(2 * t, bufs[0], bufs[1])
        half(2 * t + 1, bufs[1], bufs[0])
        return carry
    lax.fori_loop(0, (nchf + 1) // 2, pair, 0)
    pltpu.sync_copy(acc.at[pl.ds(0, NPW)], outp.at[pl.ds(nstart, NPW)])


_sc_scatter = pl.kernel(
    _sc_scatter_body,
    out_type=jax.ShapeDtypeStruct((NW * NPW, DW), jnp.float32),
    mesh=plsc.VectorSubcoreMesh(core_axis_name="c", subcore_axis_name="s"),
    compiler_params=pltpu.CompilerParams(
        needs_layout_passes=False, use_tc_tiling_on_sc=False),
    scratch_types=[
        pltpu.VMEM((3 * L,), jnp.int32),
        pltpu.VMEM((CHUNK,), jnp.int32),
        pltpu.VMEM((CHUNK,), jnp.int32),
        pltpu.VMEM((CHUNK,), jnp.float32),
        pltpu.VMEM((CHUNK,), jnp.int32),
        pltpu.VMEM((CHUNK,), jnp.int32),
        pltpu.VMEM((CHUNK,), jnp.float32),
        pltpu.VMEM((CHUNK, DW), jnp.float32),
        pltpu.VMEM((CHUNK, DW), jnp.float32),
        pltpu.VMEM((CHUNK,), jnp.float32),
        pltpu.VMEM((CHUNK,), jnp.int32),
        pltpu.VMEM((ACCR, DW), jnp.float32),
        pltpu.SemaphoreType.DMA,
        pltpu.SemaphoreType.DMA,
    ],
)


def _node_fin_body(acc_ref, exloop_ref, xl_ref, gatb_ref, bng_ref,
                   bnb_ref, lnw_ref, lnb_ref, x_ref):
    exloop = exloop_ref[...]
    xl = xl_ref[...]
    a = acc_ref[...]
    num = a[:, :D] + exloop * xl
    den = jnp.sum(a[:, D:], axis=1, keepdims=True) + exloop + 1e-16
    h = num / den + gatb_ref[...]
    mu = jnp.mean(h, axis=0, keepdims=True)
    var = jnp.mean((h - mu) ** 2, axis=0, keepdims=True)
    h = (h - mu) * jax.lax.rsqrt(var + 1e-5) * bng_ref[...] + bnb_ref[...]
    h = jnp.maximum(h, 0.0)
    x_ref[...] = jnp.maximum(h @ lnw_ref[...] + lnb_ref[...], 0.0)


def _node_fin(acc, exloop, xl, gatb, bng, bnb, lnw, lnb):
    n = xl.shape[0]
    return pl.pallas_call(
        _node_fin_body,
        out_shape=jax.ShapeDtypeStruct((n, D), jnp.float32),
    )(acc, exloop, xl, gatb[None, :], bng[None, :],
      bnb[None, :], lnw, lnb)


def _pool_mlp_body(x_ref, batch_ref, w1_ref, b1_ref, w2_ref, b2_ref,
                   w3_ref, b3_ref, out_ref):
    g = 16
    batch = batch_ref[...]
    onehot = (batch == jax.lax.broadcasted_iota(jnp.int32, (1, g), 1)
              ).astype(jnp.float32)
    sums = jnp.einsum('ng,nd->gd', onehot, x_ref[...],
                      preferred_element_type=jnp.float32)
    cnts = jnp.sum(onehot, axis=0)
    gm = sums / jnp.maximum(cnts, 1.0)[:, None]
    gm = jnp.maximum(gm @ w1_ref[...] + b1_ref[...], 0.0)
    gm = jnp.maximum(gm @ w2_ref[...] + b2_ref[...], 0.0)
    out_ref[...] = gm @ w3_ref[...] + b3_ref[...]


def _pool_mlp(x, batch, m):
    return pl.pallas_call(
        _pool_mlp_body,
        out_shape=jax.ShapeDtypeStruct((16, 1), jnp.float32),
    )(x, batch[:, None], m['W1'], m['b1'][None, :], m['W2'],
      m['b2'][None, :], m['W3'], m['b3'][None, :])


def kernel(x, edge_index, edge_attr, batch, params):
    n = x.shape[0]
    order = jnp.argsort(edge_index[1])
    src = edge_index[0][order]
    dst = edge_index[1][order]
    src2 = src.reshape(NCHT, CHUNK)
    dst2 = dst.reshape(NCHT, CHUNK)
    ea = edge_attr[order]
    nb = jnp.arange(NW + 1, dtype=jnp.int32) * NPW
    offs = jnp.pad(jnp.searchsorted(dst, nb).astype(jnp.int32),
                   (0, 3 * L - (NW + 1)))
    ones16 = jnp.concatenate(
        [jnp.ones((n, 1), jnp.float32), jnp.zeros((n, 15), jnp.float32)], 1)
    for i in range(3):
        p = params['layer%d' % i]
        w1 = p['emlp_W1']
        w1s, w1d, w1e = w1[:D], w1[D:2 * D], w1[2 * D:]
        ve = p['gat_W_edge'] @ p['gat_att_edge']
        xs1, xd1, xl, a_src, a_dst, exloop = _node_pre(
            x, w1s, w1d, p['emlp_b1'], p['gat_W'], p['gat_att_src'],
            p['gat_att_dst'])
        gs, sa = _sc_gather(xs1, xd1, a_src, a_dst, src2, dst2)
        ea, ex = _edge_dense(gs, ea, sa, w1e, p['emlp_W2'], p['emlp_b2'],
                             ve, p['le_W'], p['le_b'])
        xlp = jnp.concatenate([xl, ones16], axis=1)
        acc = _sc_scatter(xlp, ex, src2, dst2, offs)
        x = _node_fin(acc[:n], exloop, xl, p['gat_b'],
                      p['bn_g'], p['bn_b'], p['ln_W'], p['ln_b'])
    return _pool_mlp(x, batch, params['mlp'])
